# baseline (speedup 1.0000x reference)
import functools
import math

import jax
import jax.numpy as jnp
from jax import lax
from jax.experimental import pallas as pl
from jax.experimental.pallas import tpu as pltpu
from jax.experimental.pallas import tpu_sc as plsc

D_MODEL_K = 1024
VOCAB_K = 100000
SCALE_K = math.sqrt(D_MODEL_K)

_INFO = plsc.get_sparse_core_info()
_NC = _INFO.num_cores
_NS = _INFO.num_subcores
_NW = _NC * _NS
_LANES = _INFO.num_lanes

_CHUNK = 32


def _emb_kernel(table_hbm, idx_hbm, out_hbm, idx_v, rows_v, sem,
                *, b_per_w, n_chunks):
  wid = lax.axis_index("s") * _NC + lax.axis_index("c")
  base = wid * b_per_w
  pltpu.sync_copy(idx_hbm.at[pl.ds(base, b_per_w)], idx_v)

  def chunk_body(c, _):
    pltpu.async_copy(
        table_hbm.at[idx_v.at[pl.ds(c * _CHUNK, _CHUNK)]], rows_v, sem
    ).wait()

    def row_body(r, _):
      def vec_body(j, _):
        rows_v[r, pl.ds(j * _LANES, _LANES)] = (
            rows_v[r, pl.ds(j * _LANES, _LANES)] * jnp.float32(SCALE_K)
        )
        return 0
      return lax.fori_loop(0, D_MODEL_K // _LANES, vec_body, 0)

    lax.fori_loop(0, _CHUNK, row_body, 0)

    pltpu.sync_copy(rows_v, out_hbm.at[pl.ds(base + c * _CHUNK, _CHUNK)])
    return 0

  lax.fori_loop(0, n_chunks, chunk_body, 0)


def kernel(x, table):
  batch, seq = x.shape
  n_tokens = batch * seq
  d_model = table.shape[1]
  assert n_tokens % (_NW * _CHUNK) == 0
  b_per_w = n_tokens // _NW
  n_chunks = b_per_w // _CHUNK

  idx = x.reshape(n_tokens).astype(jnp.int32)

  mesh = plsc.VectorSubcoreMesh(core_axis_name="c", subcore_axis_name="s")
  run = pl.kernel(
      functools.partial(_emb_kernel, b_per_w=b_per_w, n_chunks=n_chunks),
      mesh=mesh,
      out_type=jax.ShapeDtypeStruct((n_tokens, d_model), jnp.float32),
      scratch_types=[
          pltpu.VMEM((b_per_w,), jnp.int32),
          pltpu.VMEM((_CHUNK, d_model), jnp.float32),
          pltpu.SemaphoreType.DMA,
      ],
  )
  out = run(table, idx)
  return out.reshape(batch, seq, d_model)


# speedup vs baseline: 2.3310x; 2.3310x over previous
import functools
import math

import jax
import jax.numpy as jnp
from jax import lax
from jax.experimental import pallas as pl
from jax.experimental.pallas import tpu as pltpu
from jax.experimental.pallas import tpu_sc as plsc

D_MODEL_K = 1024
SCALE_K = math.sqrt(D_MODEL_K)

_INFO = plsc.get_sparse_core_info()
_NC = _INFO.num_cores
_NS = _INFO.num_subcores
_NW = _NC * _NS
_LANES = _INFO.num_lanes

_CHUNK = 32
_NBUF = 2


def _scale_buf(rows_ref):
  def row_body(r, _):
    for j in range(D_MODEL_K // _LANES):
      rows_ref[r, pl.ds(j * _LANES, _LANES)] = (
          rows_ref[r, pl.ds(j * _LANES, _LANES)] * jnp.float32(SCALE_K)
      )
    return 0
  lax.fori_loop(0, _CHUNK, row_body, 0)


def _emb_kernel(table_hbm, idx_hbm, out_hbm, idx_v, rows_a, rows_b,
                gsem_a, gsem_b, ssem_a, ssem_b, *, b_per_w, n_chunks):
  wid = lax.axis_index("s") * _NC + lax.axis_index("c")
  base = wid * b_per_w
  pltpu.sync_copy(idx_hbm.at[pl.ds(base, b_per_w)], idx_v)

  bufs = (rows_a, rows_b)
  gsems = (gsem_a, gsem_b)
  ssems = (ssem_a, ssem_b)

  def gather_start(c, buf, gsem):
    return pltpu.async_copy(
        table_hbm.at[idx_v.at[pl.ds(c * _CHUNK, _CHUNK)]], buf, gsem)

  def store_start(c, buf, ssem):
    return pltpu.async_copy(
        buf, out_hbm.at[pl.ds(base + c * _CHUNK, _CHUNK)], ssem)

  def wait_gather(b):
    pltpu.make_async_copy(
        table_hbm.at[pl.ds(0, _CHUNK)], bufs[b], gsems[b]).wait()

  def wait_store(b):
    pltpu.make_async_copy(
        bufs[b], out_hbm.at[pl.ds(base, _CHUNK)], ssems[b]).wait()

  gather_start(0, bufs[0], gsems[0])

  def pair_body(g, _):
    for b in range(_NBUF):
      i = _NBUF * g + b
      nxt = (b + 1) % _NBUF
      wait_gather(b)
      _scale_buf(bufs[b])
      store_start(i, bufs[b], ssems[b])

      @pl.when(i + 1 < n_chunks)
      def _():
        @pl.when(i >= 1)
        def _():
          wait_store(nxt)
        gather_start(i + 1, bufs[nxt], gsems[nxt])
    return 0

  lax.fori_loop(0, n_chunks // _NBUF, pair_body, 0)
  for b in range(_NBUF):
    wait_store(b)


def kernel(x, table):
  batch, seq = x.shape
  n_tokens = batch * seq
  d_model = table.shape[1]
  assert n_tokens % (_NW * _CHUNK * _NBUF) == 0
  b_per_w = n_tokens // _NW
  n_chunks = b_per_w // _CHUNK

  idx = x.reshape(n_tokens).astype(jnp.int32)

  mesh = plsc.VectorSubcoreMesh(core_axis_name="c", subcore_axis_name="s")
  run = pl.kernel(
      functools.partial(_emb_kernel, b_per_w=b_per_w, n_chunks=n_chunks),
      mesh=mesh,
      out_type=jax.ShapeDtypeStruct((n_tokens, d_model), jnp.float32),
      scratch_types=[
          pltpu.VMEM((b_per_w,), jnp.int32),
          pltpu.VMEM((_CHUNK, d_model), jnp.float32),
          pltpu.VMEM((_CHUNK, d_model), jnp.float32),
          pltpu.SemaphoreType.DMA,
          pltpu.SemaphoreType.DMA,
          pltpu.SemaphoreType.DMA,
          pltpu.SemaphoreType.DMA,
      ],
  )
  out = run(table, idx)
  return out.reshape(batch, seq, d_model)


# speedup vs baseline: 3.0833x; 1.3228x over previous
import functools
import math

import jax
import jax.numpy as jnp
from jax import lax
from jax.experimental import pallas as pl
from jax.experimental.pallas import tpu as pltpu
from jax.experimental.pallas import tpu_sc as plsc

D_MODEL_K = 1024
SCALE_K = math.sqrt(D_MODEL_K)

_INFO = plsc.get_sparse_core_info()
_NC = _INFO.num_cores
_NS = _INFO.num_subcores
_NW = _NC * _NS
_LANES = _INFO.num_lanes

_CHUNK = 16
_NBUF = 4


def _scale_buf(rows_ref):
  def row_body(r, _):
    for j in range(D_MODEL_K // _LANES):
      rows_ref[r, pl.ds(j * _LANES, _LANES)] = (
          rows_ref[r, pl.ds(j * _LANES, _LANES)] * jnp.float32(SCALE_K)
      )
    return 0
  lax.fori_loop(0, _CHUNK, row_body, 0)


def _emb_kernel(table_hbm, idx_hbm, out_hbm, idx_v, *bufs_and_sems,
                b_per_w, n_chunks):
  bufs = bufs_and_sems[:_NBUF]
  gsems = bufs_and_sems[_NBUF:2 * _NBUF]
  ssems = bufs_and_sems[2 * _NBUF:3 * _NBUF]

  wid = lax.axis_index("s") * _NC + lax.axis_index("c")
  base = wid * b_per_w
  pltpu.sync_copy(idx_hbm.at[pl.ds(base, b_per_w)], idx_v)

  def gather_start(c, b):
    pltpu.async_copy(
        table_hbm.at[idx_v.at[pl.ds(c * _CHUNK, _CHUNK)]], bufs[b], gsems[b])

  def store_start(c, b):
    pltpu.async_copy(
        bufs[b], out_hbm.at[pl.ds(base + c * _CHUNK, _CHUNK)], ssems[b])

  def wait_gather(b):
    pltpu.make_async_copy(
        table_hbm.at[pl.ds(0, _CHUNK)], bufs[b], gsems[b]).wait()

  def wait_store(b):
    pltpu.make_async_copy(
        bufs[b], out_hbm.at[pl.ds(base, _CHUNK)], ssems[b]).wait()

  for c in range(_NBUF - 1):
    gather_start(c, c)

  def ring_body(g, _):
    for b in range(_NBUF):
      i = _NBUF * g + b
      nxt = (b + _NBUF - 1) % _NBUF
      wait_gather(b)

      @pl.when(i + _NBUF - 1 < n_chunks)
      def _():
        @pl.when(i >= 1)
        def _():
          wait_store(nxt)
        gather_start(i + _NBUF - 1, nxt)

      _scale_buf(bufs[b])
      store_start(i, b)
    return 0

  lax.fori_loop(0, n_chunks // _NBUF, ring_body, 0)
  for b in range(_NBUF):
    wait_store(b)


def kernel(x, table):
  batch, seq = x.shape
  n_tokens = batch * seq
  d_model = table.shape[1]
  assert n_tokens % (_NW * _CHUNK * _NBUF) == 0
  b_per_w = n_tokens // _NW
  n_chunks = b_per_w // _CHUNK

  idx = x.reshape(n_tokens).astype(jnp.int32)

  mesh = plsc.VectorSubcoreMesh(core_axis_name="c", subcore_axis_name="s")
  run = pl.kernel(
      functools.partial(_emb_kernel, b_per_w=b_per_w, n_chunks=n_chunks),
      mesh=mesh,
      out_type=jax.ShapeDtypeStruct((n_tokens, d_model), jnp.float32),
      scratch_types=(
          [pltpu.VMEM((b_per_w,), jnp.int32)]
          + [pltpu.VMEM((_CHUNK, d_model), jnp.float32)] * _NBUF
          + [pltpu.SemaphoreType.DMA] * (2 * _NBUF)
      ),
  )
  out = run(table, idx)
  return out.reshape(batch, seq, d_model)


# speedup vs baseline: 3.5971x; 1.1667x over previous
import functools
import math

import jax
import jax.numpy as jnp
from jax import lax
from jax.experimental import pallas as pl
from jax.experimental.pallas import tpu as pltpu
from jax.experimental.pallas import tpu_sc as plsc

D_MODEL_K = 1024
SCALE_K = math.sqrt(D_MODEL_K)

_INFO = plsc.get_sparse_core_info()
_NC = _INFO.num_cores
_NS = _INFO.num_subcores
_NW = _NC * _NS
_LANES = _INFO.num_lanes

_CHUNK = 8
_NBUF = 8


def _scale_buf(rows_ref):
  def row_body(r, _):
    for j in range(D_MODEL_K // _LANES):
      rows_ref[r, pl.ds(j * _LANES, _LANES)] = (
          rows_ref[r, pl.ds(j * _LANES, _LANES)] * jnp.float32(SCALE_K)
      )
    return 0
  lax.fori_loop(0, _CHUNK, row_body, 0)


def _emb_kernel(table_hbm, idx_hbm, out_hbm, idx_v, *bufs_and_sems,
                b_per_w, n_chunks):
  bufs = bufs_and_sems[:_NBUF]
  gsems = bufs_and_sems[_NBUF:2 * _NBUF]
  ssems = bufs_and_sems[2 * _NBUF:3 * _NBUF]

  wid = lax.axis_index("s") * _NC + lax.axis_index("c")
  base = wid * b_per_w
  pltpu.sync_copy(idx_hbm.at[pl.ds(base, b_per_w)], idx_v)

  def gather_start(c, b):
    pltpu.async_copy(
        table_hbm.at[idx_v.at[pl.ds(c * _CHUNK, _CHUNK)]], bufs[b], gsems[b])

  def store_start(c, b):
    pltpu.async_copy(
        bufs[b], out_hbm.at[pl.ds(base + c * _CHUNK, _CHUNK)], ssems[b])

  def wait_gather(b):
    pltpu.make_async_copy(
        table_hbm.at[pl.ds(0, _CHUNK)], bufs[b], gsems[b]).wait()

  def wait_store(b):
    pltpu.make_async_copy(
        bufs[b], out_hbm.at[pl.ds(base, _CHUNK)], ssems[b]).wait()

  for c in range(_NBUF - 1):
    gather_start(c, c)

  def ring_body(g, _):
    for b in range(_NBUF):
      i = _NBUF * g + b
      nxt = (b + _NBUF - 1) % _NBUF
      wait_gather(b)

      @pl.when(i + _NBUF - 1 < n_chunks)
      def _():
        @pl.when(i >= 1)
        def _():
          wait_store(nxt)
        gather_start(i + _NBUF - 1, nxt)

      _scale_buf(bufs[b])
      store_start(i, b)
    return 0

  lax.fori_loop(0, n_chunks // _NBUF, ring_body, 0)
  for b in range(_NBUF):
    wait_store(b)


def kernel(x, table):
  batch, seq = x.shape
  n_tokens = batch * seq
  d_model = table.shape[1]
  assert n_tokens % (_NW * _CHUNK * _NBUF) == 0
  b_per_w = n_tokens // _NW
  n_chunks = b_per_w // _CHUNK

  idx = x.reshape(n_tokens).astype(jnp.int32)

  mesh = plsc.VectorSubcoreMesh(core_axis_name="c", subcore_axis_name="s")
  run = pl.kernel(
      functools.partial(_emb_kernel, b_per_w=b_per_w, n_chunks=n_chunks),
      mesh=mesh,
      out_type=jax.ShapeDtypeStruct((n_tokens, d_model), jnp.float32),
      scratch_types=(
          [pltpu.VMEM((b_per_w,), jnp.int32)]
          + [pltpu.VMEM((_CHUNK, d_model), jnp.float32)] * _NBUF
          + [pltpu.SemaphoreType.DMA] * (2 * _NBUF)
      ),
  )
  out = run(table, idx)
  return out.reshape(batch, seq, d_model)
